# baseline (speedup 1.0000x reference)
import functools

import jax
import jax.numpy as jnp
from jax.experimental import pallas as pl
from jax.experimental.pallas import tpu as pltpu

_C = 21
_BLKN = 128


def _body(x_ref, lab_ref, reg_ref, tgt_ref, out_ref, acc_ref, *, n_rows, n_cols):
    i = pl.program_id(0)

    lab = lab_ref[...]
    nloc = jax.lax.broadcasted_iota(jnp.int32, lab.shape, 1)
    valid = i * _BLKN + nloc < n_cols
    x0 = x_ref[0]
    m = x0
    idx = jnp.zeros(x0.shape, jnp.int32)
    picked = jnp.where(lab == 0, x0, 0.0)
    for c in range(1, _C):
        xc = x_ref[c]
        gt = xc > m
        m = jnp.where(gt, xc, m)
        idx = jnp.where(gt, c, idx)
        picked = jnp.where(lab == c, xc, picked)
    se = jnp.exp(x0 - m)
    for c in range(1, _C):
        se += jnp.exp(x_ref[c] - m)
    lse = jnp.log(se) + m
    cls_part = jnp.sum(jnp.where(valid, lse - picked, 0.0))

    maskb = (idx > 0) & valid
    cnt_part = jnp.sum(maskb.astype(jnp.float32))

    regelem = jnp.zeros(x0.shape, jnp.float32)
    for j in range(4):
        tj = tgt_ref[:, j, :]
        mj = reg_ref[j]
        for c in range(1, _C):
            mj = jnp.where(idx == c, reg_ref[4 * c + j], mj)
        d = mj - tj
        ad = jnp.abs(d)
        regelem += jnp.where(ad < 1.0, 0.5 * d * d, ad - 0.5)
    reg_part = jnp.sum(jnp.where(maskb, regelem, 0.0))

    @pl.when(i == 0)
    def _():
        acc_ref[0] = 0.0
        acc_ref[1] = 0.0
        acc_ref[2] = 0.0

    acc_ref[0] += cls_part
    acc_ref[1] += cnt_part
    acc_ref[2] += reg_part

    @pl.when(i == pl.num_programs(0) - 1)
    def _():
        cls_loss = acc_ref[0] / n_rows
        cnt = acc_ref[1]
        reg_loss = jnp.where(
            cnt > 0.0, acc_ref[2] / jnp.maximum(cnt * 4.0, 1.0), 0.0
        )
        out_ref[0, 0] = cls_loss + reg_loss


def kernel(class_logits, bbox_reg, labels, bbox_reg_targets):
    B, N, C = class_logits.shape
    R = B * N
    xt = jnp.transpose(class_logits, (2, 0, 1))
    regt = jnp.transpose(bbox_reg, (2, 0, 1))
    tgtt = jnp.transpose(bbox_reg_targets, (0, 2, 1))
    lab = labels.astype(jnp.int32)

    grid = pl.cdiv(N, _BLKN)
    out = pl.pallas_call(
        functools.partial(_body, n_rows=float(R), n_cols=N),
        grid=(grid,),
        in_specs=[
            pl.BlockSpec((C, B, _BLKN), lambda i: (0, 0, i)),
            pl.BlockSpec((B, _BLKN), lambda i: (0, i)),
            pl.BlockSpec((4 * C, B, _BLKN), lambda i: (0, 0, i)),
            pl.BlockSpec((B, 4, _BLKN), lambda i: (0, 0, i)),
        ],
        out_specs=pl.BlockSpec(memory_space=pltpu.SMEM),
        out_shape=jax.ShapeDtypeStruct((1, 1), jnp.float32),
        scratch_shapes=[pltpu.SMEM((4,), jnp.float32)],
        compiler_params=pltpu.CompilerParams(
            dimension_semantics=("arbitrary",),
        ),
    )(xt, lab, regt, tgtt)
    return out.reshape(())


# speedup vs baseline: 2.7146x; 2.7146x over previous
import functools

import jax
import jax.numpy as jnp
from jax.experimental import pallas as pl
from jax.experimental.pallas import tpu as pltpu

_C = 21
_BLKN = 1024


def _body(x_ref, lab_ref, reg_ref, tgt_ref, out_ref, acc_ref, *, n_rows, n_cols):
    i = pl.program_id(0)

    lab = lab_ref[...]
    nloc = jax.lax.broadcasted_iota(jnp.int32, lab.shape, 1)
    valid = i * _BLKN + nloc < n_cols
    x0 = x_ref[0]
    m = x0
    idx = jnp.zeros(x0.shape, jnp.int32)
    picked = jnp.where(lab == 0, x0, 0.0)
    for c in range(1, _C):
        xc = x_ref[c]
        gt = xc > m
        m = jnp.where(gt, xc, m)
        idx = jnp.where(gt, c, idx)
        picked = jnp.where(lab == c, xc, picked)
    se = jnp.exp(x0 - m)
    for c in range(1, _C):
        se += jnp.exp(x_ref[c] - m)
    lse = jnp.log(se) + m
    cls_part = jnp.sum(jnp.where(valid, lse - picked, 0.0))

    maskb = (idx > 0) & valid
    cnt_part = jnp.sum(maskb.astype(jnp.float32))

    regelem = jnp.zeros(x0.shape, jnp.float32)
    for j in range(4):
        tj = tgt_ref[:, j, :]
        mj = reg_ref[j]
        for c in range(1, _C):
            mj = jnp.where(idx == c, reg_ref[4 * c + j], mj)
        d = mj - tj
        ad = jnp.abs(d)
        regelem += jnp.where(ad < 1.0, 0.5 * d * d, ad - 0.5)
    reg_part = jnp.sum(jnp.where(maskb, regelem, 0.0))

    @pl.when(i == 0)
    def _():
        acc_ref[0] = 0.0
        acc_ref[1] = 0.0
        acc_ref[2] = 0.0

    acc_ref[0] += cls_part
    acc_ref[1] += cnt_part
    acc_ref[2] += reg_part

    @pl.when(i == pl.num_programs(0) - 1)
    def _():
        cls_loss = acc_ref[0] / n_rows
        cnt = acc_ref[1]
        reg_loss = jnp.where(
            cnt > 0.0, acc_ref[2] / jnp.maximum(cnt * 4.0, 1.0), 0.0
        )
        out_ref[0, 0] = cls_loss + reg_loss


def kernel(class_logits, bbox_reg, labels, bbox_reg_targets):
    B, N, C = class_logits.shape
    R = B * N
    xt = jnp.transpose(class_logits, (2, 0, 1))
    regt = jnp.transpose(bbox_reg, (2, 0, 1))
    tgtt = jnp.transpose(bbox_reg_targets, (0, 2, 1))
    lab = labels.astype(jnp.int32)

    grid = pl.cdiv(N, _BLKN)
    out = pl.pallas_call(
        functools.partial(_body, n_rows=float(R), n_cols=N),
        grid=(grid,),
        in_specs=[
            pl.BlockSpec((C, B, _BLKN), lambda i: (0, 0, i)),
            pl.BlockSpec((B, _BLKN), lambda i: (0, i)),
            pl.BlockSpec((4 * C, B, _BLKN), lambda i: (0, 0, i)),
            pl.BlockSpec((B, 4, _BLKN), lambda i: (0, 0, i)),
        ],
        out_specs=pl.BlockSpec(memory_space=pltpu.SMEM),
        out_shape=jax.ShapeDtypeStruct((1, 1), jnp.float32),
        scratch_shapes=[pltpu.SMEM((4,), jnp.float32)],
        compiler_params=pltpu.CompilerParams(
            dimension_semantics=("arbitrary",),
        ),
    )(xt, lab, regt, tgtt)
    return out.reshape(())
